# baseline (speedup 1.0000x reference)
import jax
import jax.numpy as jnp
from jax.experimental import pallas as pl
from jax.experimental.pallas import tpu as pltpu

_TIME = 10000
_OUT = 64
_IN = 64


def _body(t_ref, mat_hbm, out_ref, sem):
    i = t_ref[0].astype(jnp.int32)
    pltpu.make_async_copy(mat_hbm.at[i], out_ref, sem).start()
    pltpu.make_async_copy(mat_hbm.at[i], out_ref, sem).wait()


@jax.jit
def _lookup(matrix, t1):
    return pl.pallas_call(
        _body,
        in_specs=[
            pl.BlockSpec(memory_space=pltpu.SMEM),
            pl.BlockSpec(memory_space=pl.ANY),
        ],
        out_specs=pl.BlockSpec(memory_space=pltpu.VMEM),
        out_shape=jax.ShapeDtypeStruct((_OUT, _IN), jnp.float32),
        scratch_shapes=[pltpu.SemaphoreType.DMA],
    )(t1, matrix)


def kernel(matrix, t):
    return _lookup(matrix, t.reshape(1))
